# Initial kernel scaffold; baseline (speedup 1.0000x reference)
import functools

import jax
import jax.numpy as jnp
from jax import lax
from jax.experimental import pallas as pl
from jax.experimental.pallas import tpu as pltpu
from jax.experimental.pallas import tpu_sc as plsc

N_NODES = 10000
D = 128
DEXT = 144
N_PAD = 10240
CHUNK = 128
N_SUBCORES = 16
ROWS_PER_TILE = N_PAD // N_SUBCORES
ROW_CHUNKS = ROWS_PER_TILE // CHUNK
CHUNKS_PER_TILE = 157
EDGES_PER_TILE = CHUNKS_PER_TILE * CHUNK
E_PAD = EDGES_PER_TILE * N_SUBCORES

BLK = 1024


def _sc_body(featx_hbm, srcp_hbm, dstp_hbm, sin_hbm, sout_hbm,
             idx_g, idx_s, rows, accum, sem):
    c = lax.axis_index("c")
    s = lax.axis_index("s")
    tile_row0 = s * ROWS_PER_TILE

    zeros16 = jnp.zeros((16,), jnp.float32)

    def zrow(i, _):
        def zcol(j, _):
            rows[i, pl.ds(j * 16, 16)] = zeros16
            return 0
        return lax.fori_loop(0, DEXT // 16, zcol, 0)

    lax.fori_loop(0, CHUNK, zrow, 0)

    def zacc(j, _):
        pltpu.sync_copy(rows, accum.at[pl.ds(tile_row0 + j * CHUNK, CHUNK)])
        return 0

    lax.fori_loop(0, ROW_CHUNKS, zacc, 0)
    plsc.subcore_barrier()

    def direction(g_hbm, sc_hbm, out_hbm):
        base = s * EDGES_PER_TILE

        def step(i, _):
            off = base + i * CHUNK
            pltpu.sync_copy(g_hbm.at[pl.ds(off, CHUNK)], idx_g)
            pltpu.async_copy(featx_hbm.at[idx_g], rows, sem).wait()
            pltpu.sync_copy(sc_hbm.at[pl.ds(off, CHUNK)], idx_s)
            pltpu.sync_copy(rows, accum.at[idx_s], add=True)
            return 0

        lax.fori_loop(0, CHUNKS_PER_TILE, step, 0)
        plsc.subcore_barrier()

        def cout(j, _):
            r0 = tile_row0 + j * CHUNK
            pltpu.sync_copy(accum.at[pl.ds(r0, CHUNK)], out_hbm.at[pl.ds(r0, CHUNK)])
            return 0

        lax.fori_loop(0, ROW_CHUNKS, cout, 0)

    @pl.when(c == 0)
    def _():
        direction(srcp_hbm, dstp_hbm, sin_hbm)

    @pl.when(c == 1)
    def _():
        direction(dstp_hbm, srcp_hbm, sout_hbm)


def _segment_sums(featx, srcp, dstp):
    mesh = plsc.VectorSubcoreMesh(core_axis_name="c", subcore_axis_name="s")
    k = pl.kernel(
        _sc_body,
        out_type=(
            jax.ShapeDtypeStruct((N_PAD, DEXT), jnp.float32),
            jax.ShapeDtypeStruct((N_PAD, DEXT), jnp.float32),
        ),
        mesh=mesh,
        scratch_types=[
            pltpu.VMEM((CHUNK,), jnp.int32),
            pltpu.VMEM((CHUNK,), jnp.int32),
            pltpu.VMEM((CHUNK, DEXT), jnp.float32),
            pltpu.VMEM_SHARED((N_PAD, DEXT), jnp.float32),
            pltpu.SemaphoreType.DMA,
        ],
    )
    return k(featx, srcp, dstp)


def _tc_body(featx, sin, sout, w_in, b_in, w_out, b_out, w_ih, w_hh,
             b_ih, b_hh, out):
    f = featx[...][:, :D]
    si = sin[...]
    so = sout[...]
    deg_i = si[:, D:D + 1]
    deg_o = so[:, D:D + 1]
    mean_i = si[:, :D] / jnp.maximum(deg_i, 1.0)
    mean_o = so[:, :D] / jnp.maximum(deg_o, 1.0)
    m_i = jnp.minimum(deg_i, 1.0)
    m_o = jnp.minimum(deg_o, 1.0)

    def dotT(x, w):
        return lax.dot_general(x, w, (((1,), (1,)), ((), ())),
                               preferred_element_type=jnp.float32)

    a_i = dotT(mean_i, w_in[...]) + m_i * b_in[...]
    a_o = dotT(mean_o, w_out[...]) + m_o * b_out[...]
    wih = w_ih[...]
    gi = dotT(a_i, wih[:, :D]) + dotT(a_o, wih[:, D:]) + b_ih[...]
    gh = dotT(f, w_hh[...]) + b_hh[...]
    r = jax.nn.sigmoid(gi[:, :D] + gh[:, :D])
    z = jax.nn.sigmoid(gi[:, D:2 * D] + gh[:, D:2 * D])
    n = jnp.tanh(gi[:, 2 * D:] + r * gh[:, 2 * D:])
    out[...] = (1.0 - z) * n + z * f


def _gru_update(featx, sin, sout, W_in, b_in, W_out, b_out, W_ih, W_hh,
                b_ih, b_hh):
    grid = N_PAD // BLK
    row_spec = lambda shape: pl.BlockSpec((BLK, shape), lambda i: (i, 0))
    full = lambda s: pl.BlockSpec(s, lambda i: (0,) * len(s))
    return pl.pallas_call(
        _tc_body,
        grid=(grid,),
        in_specs=[
            row_spec(DEXT),
            row_spec(DEXT),
            row_spec(DEXT),
            full((D, D)),
            full((1, D)),
            full((D, D)),
            full((1, D)),
            full((3 * D, 2 * D)),
            full((3 * D, D)),
            full((1, 3 * D)),
            full((1, 3 * D)),
        ],
        out_specs=row_spec(D),
        out_shape=jax.ShapeDtypeStruct((N_PAD, D), jnp.float32),
    )(featx, sin, sout, W_in, b_in.reshape(1, D), W_out,
      b_out.reshape(1, D), W_ih, W_hh, b_ih.reshape(1, 3 * D),
      b_hh.reshape(1, 3 * D))


@jax.jit
def kernel(feat, edge_index, W_in, b_in, W_out, b_out, W_ih, W_hh, b_ih, b_hh):
    n = feat.shape[0]
    src = edge_index[0].astype(jnp.int32)
    dst = edge_index[1].astype(jnp.int32)
    e = src.shape[0]
    padlen = E_PAD - e
    fill = jnp.full((padlen,), n, jnp.int32)
    srcp = jnp.concatenate([src, fill])
    dstp = jnp.concatenate([dst, fill])

    featx = jnp.zeros((N_PAD, DEXT), jnp.float32)
    featx = featx.at[:n, :D].set(feat)
    featx = featx.at[:n, D].set(1.0)

    sin, sout = _segment_sums(featx, srcp, dstp)
    hn = _gru_update(featx, sin, sout, W_in, b_in, W_out, b_out, W_ih,
                     W_hh, b_ih, b_hh)
    return hn[:n]


# speedup vs baseline: 4.8707x; 4.8707x over previous
import functools

import jax
import jax.numpy as jnp
from jax import lax
from jax.experimental import pallas as pl
from jax.experimental.pallas import tpu as pltpu
from jax.experimental.pallas import tpu_sc as plsc

N_NODES = 10000
D = 128
DEXT = 144
N_PAD = 10240
CHUNK = 128
N_SUBCORES = 16
ROWS_PER_TILE = N_PAD // N_SUBCORES
ROW_CHUNKS = ROWS_PER_TILE // CHUNK
CHUNKS_PER_TILE = 157
EDGES_PER_TILE = CHUNKS_PER_TILE * CHUNK
E_PAD = EDGES_PER_TILE * N_SUBCORES

BLK = 1024


def _sc_body(featx_hbm, srcp_hbm, dstp_hbm, sin_hbm, sout_hbm,
             idx_g, idx_s, rows, accum, sem):
    c = lax.axis_index("c")
    s = lax.axis_index("s")
    tile_row0 = s * ROWS_PER_TILE

    zeros16 = jnp.zeros((16,), jnp.float32)

    def zrow(i, _):
        def zcol(j, _):
            rows[i, pl.ds(j * 16, 16)] = zeros16
            return 0
        return lax.fori_loop(0, DEXT // 16, zcol, 0)

    lax.fori_loop(0, CHUNK, zrow, 0)

    def zacc(j, _):
        pltpu.sync_copy(rows, accum.at[pl.ds(tile_row0 + j * CHUNK, CHUNK)])
        return 0

    lax.fori_loop(0, ROW_CHUNKS, zacc, 0)
    plsc.subcore_barrier()

    def direction(g_hbm, sc_hbm, out_hbm):
        base = s * EDGES_PER_TILE

        def step(i, _):
            off = base + i * CHUNK
            pltpu.sync_copy(g_hbm.at[pl.ds(off, CHUNK)], idx_g)
            pltpu.async_copy(featx_hbm.at[idx_g], rows, sem).wait()
            pltpu.sync_copy(sc_hbm.at[pl.ds(off, CHUNK)], idx_s)
            pltpu.sync_copy(rows, accum.at[idx_s], add=True)
            return 0

        lax.fori_loop(0, CHUNKS_PER_TILE, step, 0)
        plsc.subcore_barrier()

        def cout(j, _):
            r0 = tile_row0 + j * CHUNK
            pltpu.sync_copy(accum.at[pl.ds(r0, CHUNK)], out_hbm.at[pl.ds(r0, CHUNK)])
            return 0

        lax.fori_loop(0, ROW_CHUNKS, cout, 0)

    @pl.when(c == 0)
    def _():
        direction(srcp_hbm, dstp_hbm, sin_hbm)

    @pl.when(c == 1)
    def _():
        direction(dstp_hbm, srcp_hbm, sout_hbm)


def _segment_sums(featx, srcp, dstp):
    mesh = plsc.VectorSubcoreMesh(core_axis_name="c", subcore_axis_name="s")
    k = pl.kernel(
        _sc_body,
        out_type=(
            jax.ShapeDtypeStruct((N_PAD, DEXT), jnp.float32),
            jax.ShapeDtypeStruct((N_PAD, DEXT), jnp.float32),
        ),
        mesh=mesh,
        scratch_types=[
            pltpu.VMEM((CHUNK,), jnp.int32),
            pltpu.VMEM((CHUNK,), jnp.int32),
            pltpu.VMEM((CHUNK, DEXT), jnp.float32),
            pltpu.VMEM_SHARED((N_PAD, DEXT), jnp.float32),
            pltpu.SemaphoreType.DMA,
        ],
        compiler_params=pltpu.CompilerParams(use_tc_tiling_on_sc=False),
    )
    return k(featx, srcp, dstp)


def _tc_body(featx, sin, sout, w_in, b_in, w_out, b_out, w_ih, w_hh,
             b_ih, b_hh, out):
    f = featx[...][:, :D]
    si = sin[...]
    so = sout[...]
    deg_i = si[:, D:D + 1]
    deg_o = so[:, D:D + 1]
    mean_i = si[:, :D] / jnp.maximum(deg_i, 1.0)
    mean_o = so[:, :D] / jnp.maximum(deg_o, 1.0)
    m_i = jnp.minimum(deg_i, 1.0)
    m_o = jnp.minimum(deg_o, 1.0)

    def dotT(x, w):
        return lax.dot_general(x, w, (((1,), (1,)), ((), ())),
                               preferred_element_type=jnp.float32)

    a_i = dotT(mean_i, w_in[...]) + m_i * b_in[...]
    a_o = dotT(mean_o, w_out[...]) + m_o * b_out[...]
    wih = w_ih[...]
    gi = dotT(a_i, wih[:, :D]) + dotT(a_o, wih[:, D:]) + b_ih[...]
    gh = dotT(f, w_hh[...]) + b_hh[...]
    r = jax.nn.sigmoid(gi[:, :D] + gh[:, :D])
    z = jax.nn.sigmoid(gi[:, D:2 * D] + gh[:, D:2 * D])
    n = jnp.tanh(gi[:, 2 * D:] + r * gh[:, 2 * D:])
    out[...] = (1.0 - z) * n + z * f


def _gru_update(featx, sin, sout, W_in, b_in, W_out, b_out, W_ih, W_hh,
                b_ih, b_hh):
    grid = N_PAD // BLK
    row_spec = lambda shape: pl.BlockSpec((BLK, shape), lambda i: (i, 0))
    full = lambda s: pl.BlockSpec(s, lambda i: (0,) * len(s))
    return pl.pallas_call(
        _tc_body,
        grid=(grid,),
        in_specs=[
            row_spec(DEXT),
            row_spec(DEXT),
            row_spec(DEXT),
            full((D, D)),
            full((1, D)),
            full((D, D)),
            full((1, D)),
            full((3 * D, 2 * D)),
            full((3 * D, D)),
            full((1, 3 * D)),
            full((1, 3 * D)),
        ],
        out_specs=row_spec(D),
        out_shape=jax.ShapeDtypeStruct((N_PAD, D), jnp.float32),
    )(featx, sin, sout, W_in, b_in.reshape(1, D), W_out,
      b_out.reshape(1, D), W_ih, W_hh, b_ih.reshape(1, 3 * D),
      b_hh.reshape(1, 3 * D))


@jax.jit
def kernel(feat, edge_index, W_in, b_in, W_out, b_out, W_ih, W_hh, b_ih, b_hh):
    n = feat.shape[0]
    src = edge_index[0].astype(jnp.int32)
    dst = edge_index[1].astype(jnp.int32)
    e = src.shape[0]
    padlen = E_PAD - e
    fill = jnp.full((padlen,), n, jnp.int32)
    srcp = jnp.concatenate([src, fill])
    dstp = jnp.concatenate([dst, fill])

    featx = jnp.zeros((N_PAD, DEXT), jnp.float32)
    featx = featx.at[:n, :D].set(feat)
    featx = featx.at[:n, D].set(1.0)

    sin, sout = _segment_sums(featx, srcp, dstp)
    hn = _gru_update(featx, sin, sout, W_in, b_in, W_out, b_out, W_ih,
                     W_hh, b_ih, b_hh)
    return hn[:n]
